# baseline (speedup 1.0000x reference)
import functools
import math

import jax
import jax.numpy as jnp
from jax import lax
from jax.experimental import pallas as pl
from jax.experimental.pallas import tpu as pltpu
from jax.experimental.pallas import tpu_sc as plsc

_CHUNK = 128
_NBUF = 4


@functools.lru_cache(maxsize=None)
def _build(total, V, D, n_chunks):
    info = plsc.get_sparse_core_info()
    NC, NS = info.num_cores, info.num_subcores
    NW = NC * NS
    per_w = total // NW
    scale = math.sqrt(D)
    mesh = plsc.VectorSubcoreMesh(core_axis_name="c", subcore_axis_name="s")

    @functools.partial(
        pl.kernel,
        mesh=mesh,
        out_type=jax.ShapeDtypeStruct((total, D), jnp.float32),
        compiler_params=pltpu.CompilerParams(use_tc_tiling_on_sc=False),
        scratch_types=(
            [pltpu.VMEM((n_chunks, _CHUNK), jnp.int32)]
            + [pltpu.VMEM((_CHUNK, D), jnp.float32) for _ in range(_NBUF)]
            + [pltpu.SemaphoreType.DMA for _ in range(_NBUF)]
        ),
    )
    def emb(idx_hbm, w_hbm, out_hbm, idx_v, *rest):
        bufs = rest[:_NBUF]
        sems = rest[_NBUF:]
        wid = lax.axis_index("s") * NC + lax.axis_index("c")
        row0 = wid * per_w

        pltpu.sync_copy(idx_hbm.at[wid], idx_v)

        for b in range(_NBUF):
            pltpu.async_copy(w_hbm.at[idx_v.at[b]], bufs[b], sems[b])

        def outer(o, carry):
            for b in range(_NBUF):
                chunk = o * _NBUF + b
                pltpu.make_async_copy(
                    w_hbm.at[idx_v.at[chunk]], bufs[b], sems[b]
                ).wait()

                def scale_row(i, c, _buf=bufs[b]):
                    for g in range(D // 16):
                        sl = pl.ds(g * 16, 16)
                        _buf[i, sl] = _buf[i, sl] * scale
                    return c

                lax.fori_loop(0, _CHUNK, scale_row, 0, unroll=2)

                pltpu.sync_copy(
                    bufs[b], out_hbm.at[pl.ds(row0 + chunk * _CHUNK, _CHUNK)]
                )

                nxt = chunk + _NBUF

                @pl.when(nxt < n_chunks)
                def _():
                    pltpu.async_copy(w_hbm.at[idx_v.at[nxt]], bufs[b], sems[b])

            return carry

        lax.fori_loop(0, n_chunks // _NBUF, outer, 0)

    return emb


def kernel(x, W):
    B, L = x.shape
    V, D = W.shape
    total = B * L
    info = plsc.get_sparse_core_info()
    NW = info.num_cores * info.num_subcores
    per_w = total // NW
    n_chunks = per_w // _CHUNK
    idx = x.reshape(NW, n_chunks, _CHUNK).astype(jnp.int32)
    out = _build(total, V, D, n_chunks)(idx, W)
    return out.reshape(B, L, D)
